# Initial kernel scaffold; baseline (speedup 1.0000x reference)
import jax
import jax.numpy as jnp
from jax.experimental import pallas as pl


def kernel(token_sequences, embedding_weight, positional_embedding):
    raise NotImplementedError("write your pallas kernel here")


# speedup vs baseline: 3.1154x; 3.1154x over previous
import functools
import jax
import jax.numpy as jnp
from jax import lax
from jax.experimental import pallas as pl
from jax.experimental.pallas import tpu as pltpu
from jax.experimental.pallas import tpu_sc as plsc

NC, NS, L = 2, 16, 16
NW = NC * NS
D = 64
BATCH, SEQ = 4096, 200
TOTAL = BATCH * SEQ
RPW = TOTAL // NW
C = SEQ
K = 2
CK = C // K
G = RPW // C

_mesh = plsc.VectorSubcoreMesh(core_axis_name="c", subcore_axis_name="s")


@functools.partial(
    pl.kernel,
    out_type=jax.ShapeDtypeStruct((TOTAL, D), jnp.float32),
    mesh=_mesh,
    scratch_types=[
        pltpu.VMEM((K, CK), jnp.int32),
        pltpu.VMEM((C, D), jnp.float32),
        pltpu.VMEM((C, D), jnp.float32),
        pltpu.VMEM((C, D), jnp.float32),
        pltpu.SemaphoreType.DMA,
    ],
    compiler_params=pltpu.CompilerParams(use_tc_tiling_on_sc=False),
)
def _emb_kernel(tok_hbm, table_hbm, pe_hbm, out_hbm,
                idx_v, rows_v, out_v, pe_v, gsem):
    wid = lax.axis_index("s") * NC + lax.axis_index("c")
    pltpu.sync_copy(pe_hbm.at[pl.ds(0, C)], pe_v)

    def chunk(g, _):
        pltpu.sync_copy(tok_hbm.at[wid, pl.ds(g * K, K)], idx_v)
        cps = [
            pltpu.async_copy(table_hbm.at[idx_v.at[j]],
                             rows_v.at[pl.ds(j * CK, CK)], gsem)
            for j in range(K)
        ]
        for cp in cps:
            cp.wait()

        def row(i, _):
            for v in range(D // L):
                sl = pl.ds(v * L, L)
                out_v[i, sl] = rows_v[i, sl] * 8.0 + pe_v[i, sl]
            return 0
        lax.fori_loop(0, C, row, 0)

        pltpu.sync_copy(out_v, out_hbm.at[pl.ds(wid * RPW + g * C, C)])
        return 0

    lax.fori_loop(0, G, chunk, 0)


def kernel(token_sequences, embedding_weight, positional_embedding):
    tok = token_sequences.reshape(NW, RPW // CK, CK)
    pe = positional_embedding.reshape(positional_embedding.shape[1], D)
    out = _emb_kernel(tok, embedding_weight, pe)
    return out.reshape(BATCH, SEQ, D)


# speedup vs baseline: 4.1416x; 1.3294x over previous
import functools
import jax
import jax.numpy as jnp
from jax import lax
from jax.experimental import pallas as pl
from jax.experimental.pallas import tpu as pltpu
from jax.experimental.pallas import tpu_sc as plsc

NC, NS, L = 2, 16, 16
NW = NC * NS
D = 64
BATCH, SEQ = 4096, 200
TOTAL = BATCH * SEQ
RPW = TOTAL // NW
C = SEQ
K = 2
CK = C // K
G = RPW // C
NBUF = 2

_mesh = plsc.VectorSubcoreMesh(core_axis_name="c", subcore_axis_name="s")


@functools.partial(
    pl.kernel,
    out_type=jax.ShapeDtypeStruct((TOTAL, D), jnp.float32),
    mesh=_mesh,
    scratch_types=[
        pltpu.VMEM((NBUF, K, CK), jnp.int32),
        pltpu.VMEM((NBUF, C, D), jnp.float32),
        pltpu.VMEM((NBUF, C, D), jnp.float32),
        pltpu.VMEM((C, D), jnp.float32),
        pltpu.SemaphoreType.DMA,
        pltpu.SemaphoreType.DMA,
        pltpu.SemaphoreType.DMA,
    ],
    compiler_params=pltpu.CompilerParams(use_tc_tiling_on_sc=False),
)
def _emb_kernel(tok_hbm, table_hbm, pe_hbm, out_hbm,
                idx_v, rows_v, out_v, pe_v, gsem, ssem, isem):
    wid = lax.axis_index("s") * NC + lax.axis_index("c")
    base = wid * RPW
    pltpu.sync_copy(pe_hbm.at[pl.ds(0, C)], pe_v)

    def fire_gathers(g, b):
        for j in range(K):
            pltpu.async_copy(table_hbm.at[idx_v.at[b, j]],
                             rows_v.at[b, pl.ds(j * CK, CK)], gsem)

    for b in range(NBUF):
        pltpu.sync_copy(tok_hbm.at[wid, pl.ds(b * K, K)], idx_v.at[b])
        fire_gathers(b, b)

    def outer(t, _):
        for b in range(NBUF):
            g = t * NBUF + b
            pltpu.make_async_copy(
                table_hbm.at[pl.ds(0, C)], rows_v.at[b], gsem).wait()

            @pl.when(g + NBUF < G)
            def _():
                pltpu.async_copy(
                    tok_hbm.at[wid, pl.ds((g + NBUF) * K, K)],
                    idx_v.at[b], isem)

            @pl.when(g >= NBUF)
            def _():
                pltpu.make_async_copy(
                    out_v.at[b], out_hbm.at[pl.ds(base, C)], ssem).wait()

            rb, ob = rows_v.at[b], out_v.at[b]

            def row(i, _):
                for v in range(D // L):
                    sl = pl.ds(v * L, L)
                    ob[i, sl] = rb[i, sl] * 8.0 + pe_v[i, sl]
                return 0
            lax.fori_loop(0, C, row, 0)

            pltpu.async_copy(out_v.at[b],
                             out_hbm.at[pl.ds(base + g * C, C)], ssem)

            @pl.when(g + NBUF < G)
            def _():
                pltpu.make_async_copy(
                    tok_hbm.at[wid, pl.ds(0, K)], idx_v.at[b], isem).wait()
                fire_gathers(g + NBUF, b)
        return 0

    lax.fori_loop(0, G // NBUF, outer, 0)

    for b in range(NBUF):
        pltpu.make_async_copy(
            out_v.at[b], out_hbm.at[pl.ds(base, C)], ssem).wait()


def kernel(token_sequences, embedding_weight, positional_embedding):
    tok = token_sequences.reshape(NW, RPW // CK, CK)
    pe = positional_embedding.reshape(positional_embedding.shape[1], D)
    out = _emb_kernel(tok, embedding_weight, pe)
    return out.reshape(BATCH, SEQ, D)
